# baseline (speedup 1.0000x reference)
import functools

import jax
import jax.numpy as jnp
from jax import lax
from jax.experimental import pallas as pl
from jax.experimental.pallas import tpu as pltpu
from jax.experimental.pallas import tpu_sc as plsc

_POS_RFACTOR = 8
_POS_SHIFT = 3
_RPAD = 32
_NB = 2
_NC, _NS, _L = 2, 16, 16


def _sc_expand(idx_hbm, out_hbm, idx_v, shift_v, out_v):
    wid = lax.axis_index("s") * _NC + lax.axis_index("c")
    b_tot = idx_hbm.shape[0]
    hp = idx_hbm.shape[1]
    rep = out_hbm.shape[-1] // hp
    rows_per_w = b_tot // (_NC * _NS)
    for rr in range(rows_per_w):
        b = wid * rows_per_w + rr
        pltpu.sync_copy(idx_hbm.at[b], idx_v)
        for c in range(hp // _L):
            shift_v[pl.ds(c * _L, _L)] = lax.shift_right_logical(
                idx_v[pl.ds(c * _L, _L)], _POS_SHIFT)

        for c in range(hp // _L):
            vec = shift_v[pl.ds(c * _L, _L)]
            for lane in range(_L):
                splat = jnp.zeros((_L,), jnp.int32) + vec[lane]
                p = c * _L + lane
                for k in range(rep // _L):
                    out_v[pl.ds(p * rep + k * _L, _L)] = splat
        pltpu.sync_copy(out_v, out_hbm.at[b])


def _embed_add_kernel(idxh_ref, tabT_ref, x_ref, o_ref):
    hn = x_ref.shape[-1]
    iota_r = jax.lax.broadcasted_iota(jnp.int32, (_RPAD, hn), 0)
    for k in range(_NB):
        m = (iota_r == idxh_ref[k]).astype(jnp.float32)
        emb = jnp.dot(tabT_ref[...], m, preferred_element_type=jnp.float32)
        o_ref[k] = x_ref[k] + emb


def kernel(x, pos, table):
    b, d, h = x.shape
    hp = pos.shape[2]
    rows = table.shape[0]
    idx_src = pos[0, :, :, 0].astype(jnp.int32)
    tab_t = jnp.zeros((d, _RPAD), jnp.float32).at[:, :rows].set(table.T)

    sc_expand = functools.partial(
        pl.kernel,
        mesh=plsc.VectorSubcoreMesh(core_axis_name="c", subcore_axis_name="s"),
        out_type=jax.ShapeDtypeStruct((b, h), jnp.int32),
        scratch_types=[
            pltpu.VMEM((hp,), jnp.int32),
            pltpu.VMEM((hp,), jnp.int32),
            pltpu.VMEM((h,), jnp.int32),
        ],
    )(_sc_expand)
    idx_h = sc_expand(idx_src).reshape(b, 1, h)

    return pl.pallas_call(
        _embed_add_kernel,
        grid=(b // _NB,),
        in_specs=[
            pl.BlockSpec((_NB, 1, h), lambda bi: (bi, 0, 0)),
            pl.BlockSpec((d, _RPAD), lambda bi: (0, 0)),
            pl.BlockSpec((_NB, d, h), lambda bi: (bi, 0, 0)),
        ],
        out_specs=pl.BlockSpec((_NB, d, h), lambda bi: (bi, 0, 0)),
        out_shape=jax.ShapeDtypeStruct(x.shape, x.dtype),
    )(idx_h, tab_t, x)


# speedup vs baseline: 1.1430x; 1.1430x over previous
import jax
import jax.numpy as jnp
from jax.experimental import pallas as pl
from jax.experimental.pallas import tpu as pltpu

_POS_SHIFT = 3
_RPAD = 32
_REP_SHIFT = 6
_NB = 2


def _embed_add_kernel(codes_ref, tabT_ref, x_ref, o_ref, sel_ref):
    hn = x_ref.shape[-1]
    hp = codes_ref.shape[-1]

    @pl.when(pl.program_id(0) == 0)
    def _build_sel():
        iota_p = jax.lax.broadcasted_iota(jnp.int32, (hp, hn), 0)
        iota_h = jax.lax.broadcasted_iota(jnp.int32, (hp, hn), 1)
        sel_ref[...] = (
            iota_p == jax.lax.shift_right_logical(iota_h, _REP_SHIFT)
        ).astype(jnp.float32)

    iota_r = jax.lax.broadcasted_iota(jnp.int32, (_RPAD, hn), 0)
    for k in range(_NB):
        codes_f = codes_ref[k].astype(jnp.float32)
        expanded = jnp.dot(codes_f, sel_ref[...],
                           preferred_element_type=jnp.float32)
        row = jax.lax.shift_right_logical(expanded.astype(jnp.int32),
                                          _POS_SHIFT)
        m = (iota_r == row).astype(jnp.float32)
        emb = jnp.dot(tabT_ref[...], m, preferred_element_type=jnp.float32)
        o_ref[k] = x_ref[k] + emb


def kernel(x, pos, table):
    b, d, h = x.shape
    hp = pos.shape[2]
    rows = table.shape[0]
    codes = pos[0, :, :, 0].astype(jnp.int32).reshape(b, 1, hp)
    tab_t = jnp.zeros((d, _RPAD), jnp.float32).at[:, :rows].set(table.T)
    return pl.pallas_call(
        _embed_add_kernel,
        grid=(b // _NB,),
        in_specs=[
            pl.BlockSpec((_NB, 1, hp), lambda bi: (bi, 0, 0)),
            pl.BlockSpec((d, _RPAD), lambda bi: (0, 0)),
            pl.BlockSpec((_NB, d, h), lambda bi: (bi, 0, 0)),
        ],
        out_specs=pl.BlockSpec((_NB, d, h), lambda bi: (bi, 0, 0)),
        out_shape=jax.ShapeDtypeStruct(x.shape, x.dtype),
        scratch_shapes=[pltpu.VMEM((hp, h), jnp.float32)],
    )(codes, tab_t, x)
